# baseline (speedup 1.0000x reference)
import functools

import jax
import jax.numpy as jnp
from jax import lax
from jax.experimental import pallas as pl
from jax.experimental.pallas import tpu as pltpu
from jax.experimental.pallas import tpu_sc as plsc

N = 10000
E = 320000
D = 128
NUM_CLASSES = 40

NC = 2
NS = 16
CW = 125
NCHUNK = E // CW
CPT = NCHUNK // (NC * NS)
GRP = 40
NGRP = CPT // GRP
WB = 1000
NWB = N // WB
ZC = 40
DEGW = 16

HIGH = jax.lax.Precision.HIGHEST


@functools.cache
def _sc_mesh():
    return plsc.VectorSubcoreMesh(core_axis_name="c", subcore_axis_name="s",
                                  num_cores=NC, num_subcores=NS)


def _fill(ref, rows, width, value):
    per = width // 16

    def body(k, _):
        i = k // per
        j = k % per
        ref[i, pl.ds(j * 16, 16)] = jnp.full((16,), value, jnp.float32)
        return 0

    lax.fori_loop(0, rows * per, body, 0)


def _sc_degree_body(dst_hbm, out_hbm, didx_v, ones_v, acc, dsem0, dsem1, sem):
    c = lax.axis_index("c")
    s = lax.axis_index("s")
    wid = c * NS + s

    pltpu.sync_copy(dst_hbm.at[pl.ds(wid * CPT, CPT)], didx_v)

    _fill(ones_v, CW, DEGW, 0.0)

    @pl.when(s < NWB)
    def _():
        def zbody(k, _):
            pltpu.sync_copy(ones_v.at[pl.ds(0, ZC)],
                            acc.at[pl.ds(s * WB + k * ZC, ZC)])
            return 0

        lax.fori_loop(0, WB // ZC, zbody, 0)

    plsc.subcore_barrier()

    _fill(ones_v, CW, DEGW, 1.0)

    def body(j, _):
        pltpu.sync_copy(ones_v, acc.at[didx_v.at[j]], add=True)
        return 0

    lax.fori_loop(0, CPT, body, 0)

    plsc.subcore_barrier()

    @pl.when(s < NWB)
    def _():
        pltpu.sync_copy(acc.at[pl.ds(s * WB, WB)],
                        out_hbm.at[c, pl.ds(s * WB, WB)])


@functools.cache
def _sc_degree_call():
    return pl.kernel(
        _sc_degree_body,
        out_type=jax.ShapeDtypeStruct((NC, N, DEGW), jnp.float32),
        mesh=_sc_mesh(),
        scratch_types=[
            pltpu.VMEM((CPT, CW), jnp.int32),
            pltpu.VMEM((CW, DEGW), jnp.float32),
            pltpu.VMEM_SHARED((N, DEGW), jnp.float32),
            pltpu.SemaphoreType.DMA,
            pltpu.SemaphoreType.DMA,
            pltpu.SemaphoreType.DMA,
        ],
    )


def _sc_degree(dst):
    return _sc_degree_call()(dst)


def _sc_scatter_body(y_hbm, src_hbm, dst_hbm, out_hbm,
                     sidx_v, didx_v, rows0, rows1, acc,
                     gsem0, gsem1, ssem0, ssem1, sem):
    c = lax.axis_index("c")
    s = lax.axis_index("s")
    wid = c * NS + s

    _fill(rows0, ZC, D, 0.0)

    @pl.when(s < NWB)
    def _():
        def zbody(k, _):
            pltpu.sync_copy(rows0.at[pl.ds(0, ZC)],
                            acc.at[pl.ds(s * WB + k * ZC, ZC)])
            return 0

        lax.fori_loop(0, WB // ZC, zbody, 0)

    plsc.subcore_barrier()

    for g in range(NGRP):
        base = wid * CPT + g * GRP
        pltpu.sync_copy(src_hbm.at[pl.ds(base, GRP)], sidx_v)
        pltpu.sync_copy(dst_hbm.at[pl.ds(base, GRP)], didx_v)

        pltpu.async_copy(y_hbm.at[sidx_v.at[0]], rows0, gsem0)
        pltpu.async_copy(y_hbm.at[sidx_v.at[1]], rows1, gsem1)
        for a in range(0, GRP, 2):
            b = a + 1
            pltpu.make_async_copy(y_hbm.at[sidx_v.at[a]], rows0, gsem0).wait()
            pltpu.sync_copy(rows0, acc.at[didx_v.at[a]], add=True)
            if a + 2 < GRP:
                pltpu.async_copy(y_hbm.at[sidx_v.at[a + 2]], rows0, gsem0)
            pltpu.make_async_copy(y_hbm.at[sidx_v.at[b]], rows1, gsem1).wait()
            pltpu.sync_copy(rows1, acc.at[didx_v.at[b]], add=True)
            if b + 2 < GRP:
                pltpu.async_copy(y_hbm.at[sidx_v.at[b + 2]], rows1, gsem1)

    plsc.subcore_barrier()

    @pl.when(s < NWB)
    def _():
        pltpu.sync_copy(acc.at[pl.ds(s * WB, WB)],
                        out_hbm.at[c, pl.ds(s * WB, WB)])


@functools.cache
def _sc_scatter_call():
    return pl.kernel(
        _sc_scatter_body,
        out_type=jax.ShapeDtypeStruct((NC, N, D), jnp.float32),
        mesh=_sc_mesh(),
        scratch_types=[
            pltpu.VMEM((GRP, CW), jnp.int32),
            pltpu.VMEM((GRP, CW), jnp.int32),
            pltpu.VMEM((CW, D), jnp.float32),
            pltpu.VMEM((CW, D), jnp.float32),
            pltpu.VMEM_SHARED((N, D), jnp.float32),
            pltpu.SemaphoreType.DMA,
            pltpu.SemaphoreType.DMA,
            pltpu.SemaphoreType.DMA,
            pltpu.SemaphoreType.DMA,
            pltpu.SemaphoreType.DMA,
        ],
    )


def _sc_scatter(y, src, dst):
    return _sc_scatter_call()(y, src, dst)


R = 1000


def _tc_xw_body(x_ref, w1_ref, xw_ref):
    xw_ref[...] = jnp.dot(x_ref[...], w1_ref[...], precision=HIGH,
                          preferred_element_type=jnp.float32)


_tc_xw = pl.pallas_call(
    _tc_xw_body,
    grid=(N // R,),
    in_specs=[
        pl.BlockSpec((R, D), lambda i: (i, 0)),
        pl.BlockSpec((D, D), lambda i: (0, 0)),
    ],
    out_specs=pl.BlockSpec((R, D), lambda i: (i, 0)),
    out_shape=jax.ShapeDtypeStruct((N, D), jnp.float32),
)


def _tc_scale_body(degp_ref, xw_ref, y1_ref, dinvb_ref):
    d = degp_ref[...]
    deg = d[0, :, 0:1] + d[1, :, 0:1] + 1.0
    dinv = lax.rsqrt(deg)
    y1_ref[...] = dinv * xw_ref[...]
    dinvb_ref[...] = jnp.broadcast_to(dinv, (R, D))


_tc_scale = pl.pallas_call(
    _tc_scale_body,
    grid=(N // R,),
    in_specs=[
        pl.BlockSpec((NC, R, DEGW), lambda i: (0, i, 0)),
        pl.BlockSpec((R, D), lambda i: (i, 0)),
    ],
    out_specs=[
        pl.BlockSpec((R, D), lambda i: (i, 0)),
        pl.BlockSpec((R, D), lambda i: (i, 0)),
    ],
    out_shape=[
        jax.ShapeDtypeStruct((N, D), jnp.float32),
        jax.ShapeDtypeStruct((N, D), jnp.float32),
    ],
)


def _tc_mid_body(s1_ref, y1_ref, dinvb_ref, w2_ref, b1_ref, y2_ref):
    s1 = s1_ref[...]
    agg = dinvb_ref[...] * (s1[0] + s1[1] + y1_ref[...]) + b1_ref[...]
    h = jnp.maximum(agg, 0.0)
    xw = jnp.dot(h, w2_ref[...], precision=HIGH,
                 preferred_element_type=jnp.float32)
    y2_ref[...] = dinvb_ref[...] * xw


_tc_mid = pl.pallas_call(
    _tc_mid_body,
    grid=(N // R,),
    in_specs=[
        pl.BlockSpec((NC, R, D), lambda i: (0, i, 0)),
        pl.BlockSpec((R, D), lambda i: (i, 0)),
        pl.BlockSpec((R, D), lambda i: (i, 0)),
        pl.BlockSpec((D, D), lambda i: (0, 0)),
        pl.BlockSpec((1, D), lambda i: (0, 0)),
    ],
    out_specs=pl.BlockSpec((R, D), lambda i: (i, 0)),
    out_shape=jax.ShapeDtypeStruct((N, D), jnp.float32),
)


def _tc_head_body(s2_ref, y2_ref, dinvb_ref, b2_ref, wm1_ref, bm1_ref,
                  wm2_ref, bm2_ref, out_ref):
    s2 = s2_ref[...]
    low2 = dinvb_ref[...] * (s2[0] + s2[1] + y2_ref[...]) + b2_ref[...]
    z = jnp.maximum(
        jnp.dot(low2, wm1_ref[...], precision=HIGH,
                preferred_element_type=jnp.float32) + bm1_ref[...], 0.0)
    logits = jnp.dot(z, wm2_ref[...], precision=HIGH,
                     preferred_element_type=jnp.float32) + bm2_ref[...]
    m = jnp.max(logits, axis=1, keepdims=True)
    lse = m + jnp.log(jnp.sum(jnp.exp(logits - m), axis=1, keepdims=True))
    out_ref[...] = logits - lse


_tc_head = pl.pallas_call(
    _tc_head_body,
    grid=(N // R,),
    in_specs=[
        pl.BlockSpec((NC, R, D), lambda i: (0, i, 0)),
        pl.BlockSpec((R, D), lambda i: (i, 0)),
        pl.BlockSpec((R, D), lambda i: (i, 0)),
        pl.BlockSpec((1, D), lambda i: (0, 0)),
        pl.BlockSpec((D, D), lambda i: (0, 0)),
        pl.BlockSpec((1, D), lambda i: (0, 0)),
        pl.BlockSpec((D, NUM_CLASSES), lambda i: (0, 0)),
        pl.BlockSpec((1, NUM_CLASSES), lambda i: (0, 0)),
    ],
    out_specs=pl.BlockSpec((R, NUM_CLASSES), lambda i: (i, 0)),
    out_shape=jax.ShapeDtypeStruct((N, NUM_CLASSES), jnp.float32),
)


def kernel(x, edge_index, W_gcn1, b_gcn1, W_gcn2, b_gcn2, W_lin1, W_lin2,
           W_mlp1, b_mlp1, W_mlp2, b_mlp2):
    src = edge_index[0].reshape(NCHUNK, CW)
    dst = edge_index[1].reshape(NCHUNK, CW)

    degp = _sc_degree(dst)
    xw1 = _tc_xw(x, W_gcn1)
    y1, dinvb = _tc_scale(degp, xw1)
    s1 = _sc_scatter(y1, src, dst)
    y2 = _tc_mid(s1, y1, dinvb, W_gcn2, b_gcn1.reshape(1, D))
    s2 = _sc_scatter(y2, src, dst)
    out = _tc_head(s2, y2, dinvb, b_gcn2.reshape(1, D), W_mlp1,
                   b_mlp1.reshape(1, D), W_mlp2, b_mlp2.reshape(1, NUM_CLASSES))
    return out


# speedup vs baseline: 1.0649x; 1.0649x over previous
import functools

import jax
import jax.numpy as jnp
from jax import lax
from jax.experimental import pallas as pl
from jax.experimental.pallas import tpu as pltpu
from jax.experimental.pallas import tpu_sc as plsc

N = 10000
E = 320000
D = 128
NUM_CLASSES = 40

NC = 2
NS = 16
CW = 125
NCHUNK = E // CW
CPT = NCHUNK // (NC * NS)
GRP = 40
NGRP = CPT // GRP
WB = 1000
NWB = N // WB
ZC = 40
DEGW = 16

HIGH = jax.lax.Precision.HIGHEST


@functools.cache
def _sc_mesh():
    return plsc.VectorSubcoreMesh(core_axis_name="c", subcore_axis_name="s",
                                  num_cores=NC, num_subcores=NS)


def _fill(ref, rows, width, value):
    per = width // 16

    def body(k, _):
        i = k // per
        j = k % per
        ref[i, pl.ds(j * 16, 16)] = jnp.full((16,), value, jnp.float32)
        return 0

    lax.fori_loop(0, rows * per, body, 0)


def _sc_degree_body(dst_hbm, out_hbm, didx_v, ones_v, acc, dsem0, dsem1, sem):
    c = lax.axis_index("c")
    s = lax.axis_index("s")
    wid = c * NS + s

    pltpu.sync_copy(dst_hbm.at[pl.ds(wid * CPT, CPT)], didx_v)

    _fill(ones_v, CW, DEGW, 0.0)

    @pl.when(s < NWB)
    def _():
        def zbody(k, _):
            pltpu.sync_copy(ones_v.at[pl.ds(0, ZC)],
                            acc.at[pl.ds(s * WB + k * ZC, ZC)])
            return 0

        lax.fori_loop(0, WB // ZC, zbody, 0)

    plsc.subcore_barrier()

    _fill(ones_v, CW, DEGW, 1.0)

    def body(j, _):
        pltpu.sync_copy(ones_v, acc.at[didx_v.at[j]], add=True)
        return 0

    lax.fori_loop(0, CPT, body, 0)

    plsc.subcore_barrier()

    @pl.when(s < NWB)
    def _():
        pltpu.sync_copy(acc.at[pl.ds(s * WB, WB)],
                        out_hbm.at[c, pl.ds(s * WB, WB)])


@functools.cache
def _sc_degree_call():
    return pl.kernel(
        _sc_degree_body,
        out_type=jax.ShapeDtypeStruct((NC, N, DEGW), jnp.float32),
        mesh=_sc_mesh(),
        scratch_types=[
            pltpu.VMEM((CPT, CW), jnp.int32),
            pltpu.VMEM((CW, DEGW), jnp.float32),
            pltpu.VMEM_SHARED((N, DEGW), jnp.float32),
            pltpu.SemaphoreType.DMA,
            pltpu.SemaphoreType.DMA,
            pltpu.SemaphoreType.DMA,
        ],
    )


def _sc_degree(dst):
    return _sc_degree_call()(dst)


def _sc_scatter_body(y_hbm, src_hbm, dst_hbm, out_hbm,
                     sidx_v, didx_v, rows0, rows1, acc,
                     gsem0, gsem1, ssem0, ssem1, sem):
    c = lax.axis_index("c")
    s = lax.axis_index("s")
    wid = c * NS + s

    _fill(rows0, ZC, D, 0.0)

    @pl.when(s < NWB)
    def _():
        def zbody(k, _):
            pltpu.sync_copy(rows0.at[pl.ds(0, ZC)],
                            acc.at[pl.ds(s * WB + k * ZC, ZC)])
            return 0

        lax.fori_loop(0, WB // ZC, zbody, 0)

    plsc.subcore_barrier()

    for g in range(NGRP):
        base = wid * CPT + g * GRP
        pltpu.sync_copy(src_hbm.at[pl.ds(base, GRP)], sidx_v)
        pltpu.sync_copy(dst_hbm.at[pl.ds(base, GRP)], didx_v)

        pltpu.async_copy(y_hbm.at[sidx_v.at[0]], rows0, gsem0)
        pltpu.async_copy(y_hbm.at[sidx_v.at[1]], rows1, gsem1)
        for a in range(0, GRP, 2):
            b = a + 1
            pltpu.make_async_copy(y_hbm.at[sidx_v.at[a]], rows0, gsem0).wait()
            pltpu.sync_copy(rows0, acc.at[didx_v.at[a]], add=True)
            if a + 2 < GRP:
                pltpu.async_copy(y_hbm.at[sidx_v.at[a + 2]], rows0, gsem0)
            pltpu.make_async_copy(y_hbm.at[sidx_v.at[b]], rows1, gsem1).wait()
            pltpu.sync_copy(rows1, acc.at[didx_v.at[b]], add=True)
            if b + 2 < GRP:
                pltpu.async_copy(y_hbm.at[sidx_v.at[b + 2]], rows1, gsem1)

    plsc.subcore_barrier()

    @pl.when(s < NWB)
    def _():
        pltpu.sync_copy(acc.at[pl.ds(s * WB, WB)],
                        out_hbm.at[c, pl.ds(s * WB, WB)])


@functools.cache
def _sc_scatter_call():
    return pl.kernel(
        _sc_scatter_body,
        out_type=jax.ShapeDtypeStruct((NC, N, D), jnp.float32),
        mesh=_sc_mesh(),
        scratch_types=[
            pltpu.VMEM((GRP, CW), jnp.int32),
            pltpu.VMEM((GRP, CW), jnp.int32),
            pltpu.VMEM((CW, D), jnp.float32),
            pltpu.VMEM((CW, D), jnp.float32),
            pltpu.VMEM_SHARED((N, D), jnp.float32),
            pltpu.SemaphoreType.DMA,
            pltpu.SemaphoreType.DMA,
            pltpu.SemaphoreType.DMA,
            pltpu.SemaphoreType.DMA,
            pltpu.SemaphoreType.DMA,
        ],
    )


def _sc_scatter(y, src, dst):
    return _sc_scatter_call()(y, src, dst)


R = 2000


def _tc_xw_body(x_ref, w1_ref, xw_ref):
    xw_ref[...] = jnp.dot(x_ref[...], w1_ref[...], precision=HIGH,
                          preferred_element_type=jnp.float32)


_tc_xw = pl.pallas_call(
    _tc_xw_body,
    grid=(N // R,),
    in_specs=[
        pl.BlockSpec((R, D), lambda i: (i, 0)),
        pl.BlockSpec((D, D), lambda i: (0, 0)),
    ],
    out_specs=pl.BlockSpec((R, D), lambda i: (i, 0)),
    out_shape=jax.ShapeDtypeStruct((N, D), jnp.float32),
)


def _tc_scale_body(degp_ref, xw_ref, y1_ref, dinvb_ref):
    d = degp_ref[...]
    deg = d[0, :, 0:1] + d[1, :, 0:1] + 1.0
    dinv = lax.rsqrt(deg)
    y1_ref[...] = dinv * xw_ref[...]
    dinvb_ref[...] = jnp.broadcast_to(dinv, (R, D))


_tc_scale = pl.pallas_call(
    _tc_scale_body,
    grid=(N // R,),
    in_specs=[
        pl.BlockSpec((NC, R, DEGW), lambda i: (0, i, 0)),
        pl.BlockSpec((R, D), lambda i: (i, 0)),
    ],
    out_specs=[
        pl.BlockSpec((R, D), lambda i: (i, 0)),
        pl.BlockSpec((R, D), lambda i: (i, 0)),
    ],
    out_shape=[
        jax.ShapeDtypeStruct((N, D), jnp.float32),
        jax.ShapeDtypeStruct((N, D), jnp.float32),
    ],
)


def _tc_mid_body(s1_ref, y1_ref, dinvb_ref, w2_ref, b1_ref, y2_ref):
    s1 = s1_ref[...]
    agg = dinvb_ref[...] * (s1[0] + s1[1] + y1_ref[...]) + b1_ref[...]
    h = jnp.maximum(agg, 0.0)
    xw = jnp.dot(h, w2_ref[...], precision=HIGH,
                 preferred_element_type=jnp.float32)
    y2_ref[...] = dinvb_ref[...] * xw


_tc_mid = pl.pallas_call(
    _tc_mid_body,
    grid=(N // R,),
    in_specs=[
        pl.BlockSpec((NC, R, D), lambda i: (0, i, 0)),
        pl.BlockSpec((R, D), lambda i: (i, 0)),
        pl.BlockSpec((R, D), lambda i: (i, 0)),
        pl.BlockSpec((D, D), lambda i: (0, 0)),
        pl.BlockSpec((1, D), lambda i: (0, 0)),
    ],
    out_specs=pl.BlockSpec((R, D), lambda i: (i, 0)),
    out_shape=jax.ShapeDtypeStruct((N, D), jnp.float32),
)


def _tc_head_body(s2_ref, y2_ref, dinvb_ref, b2_ref, wm1_ref, bm1_ref,
                  wm2_ref, bm2_ref, out_ref):
    s2 = s2_ref[...]
    low2 = dinvb_ref[...] * (s2[0] + s2[1] + y2_ref[...]) + b2_ref[...]
    z = jnp.maximum(
        jnp.dot(low2, wm1_ref[...], precision=HIGH,
                preferred_element_type=jnp.float32) + bm1_ref[...], 0.0)
    logits = jnp.dot(z, wm2_ref[...], precision=HIGH,
                     preferred_element_type=jnp.float32) + bm2_ref[...]
    m = jnp.max(logits, axis=1, keepdims=True)
    lse = m + jnp.log(jnp.sum(jnp.exp(logits - m), axis=1, keepdims=True))
    out_ref[...] = logits - lse


_tc_head = pl.pallas_call(
    _tc_head_body,
    grid=(N // R,),
    in_specs=[
        pl.BlockSpec((NC, R, D), lambda i: (0, i, 0)),
        pl.BlockSpec((R, D), lambda i: (i, 0)),
        pl.BlockSpec((R, D), lambda i: (i, 0)),
        pl.BlockSpec((1, D), lambda i: (0, 0)),
        pl.BlockSpec((D, D), lambda i: (0, 0)),
        pl.BlockSpec((1, D), lambda i: (0, 0)),
        pl.BlockSpec((D, NUM_CLASSES), lambda i: (0, 0)),
        pl.BlockSpec((1, NUM_CLASSES), lambda i: (0, 0)),
    ],
    out_specs=pl.BlockSpec((R, NUM_CLASSES), lambda i: (i, 0)),
    out_shape=jax.ShapeDtypeStruct((N, NUM_CLASSES), jnp.float32),
)


def kernel(x, edge_index, W_gcn1, b_gcn1, W_gcn2, b_gcn2, W_lin1, W_lin2,
           W_mlp1, b_mlp1, W_mlp2, b_mlp2):
    src = edge_index[0].reshape(NCHUNK, CW)
    dst = edge_index[1].reshape(NCHUNK, CW)

    degp = _sc_degree(dst)
    xw1 = _tc_xw(x, W_gcn1)
    y1, dinvb = _tc_scale(degp, xw1)
    s1 = _sc_scatter(y1, src, dst)
    y2 = _tc_mid(s1, y1, dinvb, W_gcn2, b_gcn1.reshape(1, D))
    s2 = _sc_scatter(y2, src, dst)
    out = _tc_head(s2, y2, dinvb, b_gcn2.reshape(1, D), W_mlp1,
                   b_mlp1.reshape(1, D), W_mlp2, b_mlp2.reshape(1, NUM_CLASSES))
    return out
